# baseline (speedup 1.0000x reference)
import jax
import jax.numpy as jnp
from jax.experimental import pallas as pl
from jax.experimental.pallas import tpu as pltpu

CO = 410
KS = 7
POOL = 7
DIN = 109
DPAD = DIN + 2
DC = DPAD - KS + 1
NP = DC // POOL
TAPS = KS * KS * KS
WIN = 16
PW = 16
LH = 8
KHJ = 128
MROWS = NP * PW * LH
KDIM = KS * KHJ
F1I = NP * NP * NP
F1O = 800
FK = NP * NP * PW
NCLS = 2
CP = 8
NEG = -1e30
VMEM = 64 * 1024 * 1024


MTILE = 320
NTILES = MROWS // MTILE


def _conv_body(x0, x1, x2, x3, x4, x5, x6, w_ref, b_ref, o_ref):
    ld = pl.program_id(1)
    planes = (x0, x1, x2, x3, x4, x5, x6)
    p = jnp.concatenate(
        [planes[kd][0].reshape(MROWS, KHJ) for kd in range(KS)],
        axis=1)

    chunks = []
    for c in range(NTILES):
        sub = p[c * MTILE:(c + 1) * MTILE, :]
        acc = jnp.dot(sub, w_ref[0], preferred_element_type=jnp.float32)
        for lw in range(1, POOL):
            acc = jnp.maximum(
                acc, jnp.dot(sub, w_ref[lw], preferred_element_type=jnp.float32))
        a4 = acc.reshape(MTILE // LH, LH, CO)
        chunks.append(jnp.max(a4[:, :POOL, :], axis=1))
    pooled = jnp.concatenate(chunks, axis=0)

    @pl.when(ld == 0)
    def _first():
        o_ref[0] = pooled

    @pl.when(jnp.logical_and(ld > 0, ld < POOL - 1))
    def _mid():
        o_ref[0] = jnp.maximum(o_ref[0], pooled)

    @pl.when(ld == POOL - 1)
    def _last():
        o_ref[0] = jnp.maximum(o_ref[0], pooled) + b_ref[...]


def _expand_body(x_ref, o_ref):
    xp = x_ref[0]
    for kh in range(KS):
        rs = jnp.stack([xp[7 * p + kh:7 * p + kh + LH, :] for p in range(NP)],
                       axis=0)
        chunk = jnp.stack([rs[:, :, 7 * w:7 * w + WIN] for w in range(PW)],
                          axis=1)
        o_ref[0, :, :, :, WIN * kh:WIN * (kh + 1)] = chunk
    o_ref[0, :, :, :, KS * WIN:] = jnp.zeros((NP, PW, LH, KHJ - KS * WIN),
                                             jnp.bfloat16)


def _tap_table(x):
    vol = x[0, 0].astype(jnp.bfloat16)
    xp = jnp.pad(vol, ((1, 1), (1, 2), (1, 18)))
    return pl.pallas_call(
        _expand_body,
        out_shape=jax.ShapeDtypeStruct((DPAD, NP, PW, LH, KHJ), jnp.bfloat16),
        grid_spec=pltpu.PrefetchScalarGridSpec(
            num_scalar_prefetch=0,
            grid=(DPAD,),
            in_specs=[pl.BlockSpec((1, 112, KHJ), lambda d: (d, 0, 0))],
            out_specs=pl.BlockSpec((1, NP, PW, LH, KHJ),
                                   lambda d: (d, 0, 0, 0, 0)),
        ),
        compiler_params=pltpu.CompilerParams(
            dimension_semantics=("parallel",),
            vmem_limit_bytes=VMEM),
    )(xp)


def _shifted_weights(conv_w):
    wt = conv_w.reshape(CO, TAPS).T
    w4 = wt.reshape(KS * KS, KS, CO)
    mats = []
    for lw in range(POOL):
        m = jnp.pad(w4, ((0, 0), (lw, WIN - KS - lw), (0, 0)))
        m = m.reshape(KS, KS * WIN, CO)
        m = jnp.pad(m, ((0, 0), (0, KHJ - KS * WIN), (0, 0)))
        mats.append(m.reshape(KDIM, CO))
    return jnp.stack(mats, 0).astype(jnp.bfloat16)


def _fc_body(x_ref, w1_ref, b1_ref, w2_ref, b2_ref, o_ref):
    feats = x_ref[...].astype(jnp.bfloat16)
    h = jnp.dot(w1_ref[...], feats,
                preferred_element_type=jnp.float32) + b1_ref[...]
    logits = jnp.dot(w2_ref[...], h.astype(jnp.bfloat16),
                     preferred_element_type=jnp.float32) + b2_ref[...]
    m = jnp.max(logits, axis=0, keepdims=True)
    e = jnp.exp(logits - m)
    o_ref[...] = e / jnp.sum(e, axis=0, keepdims=True)


def kernel(x, conv_w, conv_b, fc1_w, fc1_b, fc2_w, fc2_b):
    xw6 = _tap_table(x)
    w3 = _shifted_weights(conv_w)
    b_r = conv_b.reshape(1, CO).astype(jnp.float32)

    in_specs = [
        pl.BlockSpec((1, NP, PW, LH, KHJ),
                     lambda pd, ld, kd=kd: (7 * pd + ld + kd, 0, 0, 0, 0))
        for kd in range(KS)
    ] + [
        pl.BlockSpec((POOL, KDIM, CO), lambda pd, ld: (0, 0, 0)),
        pl.BlockSpec((1, CO), lambda pd, ld: (0, 0)),
    ]
    conv_out = pl.pallas_call(
        _conv_body,
        out_shape=jax.ShapeDtypeStruct((NP, NP * PW, CO), jnp.float32),
        grid_spec=pltpu.PrefetchScalarGridSpec(
            num_scalar_prefetch=0,
            grid=(NP, POOL),
            in_specs=in_specs,
            out_specs=pl.BlockSpec((1, NP * PW, CO), lambda pd, ld: (pd, 0, 0)),
        ),
        compiler_params=pltpu.CompilerParams(
            dimension_semantics=("parallel", "arbitrary"),
            vmem_limit_bytes=VMEM),
    )(*([xw6] * KS), w3, b_r)

    w1r = fc1_w.reshape(F1O, NP, NP, NP)
    w1r = jnp.pad(w1r, ((0, 0), (0, 0), (0, 0), (0, PW - NP)))
    w1r = w1r.reshape(F1O, FK).astype(jnp.bfloat16)
    b1r = fc1_b.reshape(F1O, 1)
    w2p = jnp.pad(fc2_w, ((0, CP - NCLS), (0, 0))).astype(jnp.bfloat16)
    b2p = jnp.full((CP, 1), NEG, jnp.float32).at[:NCLS, 0].set(fc2_b)

    probs = pl.pallas_call(
        _fc_body,
        out_shape=jax.ShapeDtypeStruct((CP, CO), jnp.float32),
        in_specs=[
            pl.BlockSpec((FK, CO), lambda: (0, 0)),
            pl.BlockSpec((F1O, FK), lambda: (0, 0)),
            pl.BlockSpec((F1O, 1), lambda: (0, 0)),
            pl.BlockSpec((CP, F1O), lambda: (0, 0)),
            pl.BlockSpec((CP, 1), lambda: (0, 0)),
        ],
        out_specs=pl.BlockSpec((CP, CO), lambda: (0, 0)),
        compiler_params=pltpu.CompilerParams(vmem_limit_bytes=VMEM),
    )(conv_out.reshape(FK, CO), w1r, b1r, w2p, b2p)

    pooled = conv_out.reshape(NP, NP, PW, CO)[:, :, :NP, :]
    return probs[:NCLS, :].T, pooled
